# baseline (speedup 1.0000x reference)
import functools

import jax
import jax.numpy as jnp
from jax import lax
from jax.experimental import pallas as pl
from jax.experimental.pallas import tpu as pltpu
from jax.experimental.pallas import tpu_sc as plsc

_CH = 128
_SC = 256
_NBUF = 2


@functools.lru_cache(maxsize=None)
def _make_lookup(B, V, D, nc, ns):
    NW = nc * ns
    b_per_w = B // NW
    n_sc = b_per_w // _SC
    g_per_sc = _SC // _CH
    assert n_sc % _NBUF == 0 and n_sc >= 2
    mesh = plsc.VectorSubcoreMesh(core_axis_name="c", subcore_axis_name="s")

    @functools.partial(
        pl.kernel,
        mesh=mesh,
        out_type=jax.ShapeDtypeStruct((B, D), jnp.float32),
        scratch_types=[
            pltpu.VMEM_SHARED((V, D), jnp.float32),
            pltpu.VMEM((b_per_w,), jnp.int32),
            pltpu.VMEM((_NBUF, _SC, D), jnp.float32),
        ]
        + [pltpu.SemaphoreType.DMA] * (2 * _NBUF),
    )
    def lookup_kernel(idx_hbm, table_hbm, out_hbm, table_sh, idx_v, rows_v,
                      *sems):
        sem_g = sems[:_NBUF]
        sem_s = sems[_NBUF:]
        wid = lax.axis_index("s") * nc + lax.axis_index("c")
        base = wid * b_per_w

        @pl.when(lax.axis_index("s") == 0)
        def _():
            pltpu.sync_copy(table_hbm, table_sh)

        pltpu.sync_copy(idx_hbm.at[pl.ds(base, b_per_w)], idx_v)
        plsc.subcore_barrier()

        def gather_desc(s, b, h):
            idx_sl = idx_v.at[pl.ds(s * _SC + h * _CH, _CH)]
            return pltpu.make_async_copy(
                table_sh.at[idx_sl],
                rows_v.at[b].at[pl.ds(h * _CH, _CH)],
                sem_g[b])

        def scatter_desc(s, b):
            return pltpu.make_async_copy(
                rows_v.at[b],
                out_hbm.at[pl.ds(base + s * _SC, _SC)],
                sem_s[b])

        for h in range(g_per_sc):
            gather_desc(0, 0, h).start()

        def body(ss, carry):
            for u in range(_NBUF):
                s = ss * _NBUF + u
                b, nb = u, 1 - u

                @pl.when((s >= 1) & (s + 1 < n_sc))
                def _():
                    scatter_desc(s - 1, nb).wait()

                @pl.when(s + 1 < n_sc)
                def _():
                    for h in range(g_per_sc):
                        gather_desc(s + 1, nb, h).start()

                for h in range(g_per_sc):
                    gather_desc(s, b, h).wait()
                scatter_desc(s, b).start()
            return carry

        lax.fori_loop(0, n_sc // _NBUF, body, 0)
        scatter_desc(n_sc - 2, (n_sc - 2) % _NBUF).wait()
        scatter_desc(n_sc - 1, (n_sc - 1) % _NBUF).wait()

    return lookup_kernel


def kernel(elems, table):
    shape = elems.shape
    V, D = table.shape
    idx = elems.reshape(-1).astype(jnp.int32)
    B = idx.shape[0]
    info = plsc.get_sparse_core_info()
    nc, ns = info.num_cores, info.num_subcores
    group = nc * ns * _SC * _NBUF
    Bp = ((B + group - 1) // group) * group
    if Bp != B:
        idx = jnp.pad(idx, (0, Bp - B))
    out = _make_lookup(Bp, V, D, nc, ns)(idx, table)
    if Bp != B:
        out = out[:B]
    return out.reshape(*shape, D)
